# baseline (speedup 1.0000x reference)
import functools
import math

import jax
import jax.numpy as jnp
from jax import lax
from jax.experimental import pallas as pl
from jax.experimental.pallas import tpu as pltpu
from jax.experimental.pallas import tpu_sc as plsc

_NCLS = 722
_DEC = [math.exp(-(2.0 ** d) / 4.0) for d in range(4)]
_NC, _NS = 2, 16
_NW = _NC * _NS
_ROWS = 16 * 2048
_RPW = _ROWS // _NW
_RCH = 64
_NCHUNK = _RPW // _RCH
_CB, _CU = 38, 19


def _log16(x):
    bits = plsc.bitcast(x, jnp.int32)
    e = ((bits >> 23) & 0xFF) - 126
    f = plsc.bitcast((bits & 0x7FFFFF) | 0x3F000000, jnp.float32)
    ef = e.astype(jnp.float32)
    big = f > 0.70710678118654752440
    z = jnp.where(big, f - 1.0, 2.0 * f - 1.0)
    ef = jnp.where(big, ef, ef - 1.0)
    zz = z * z
    p = jnp.full((16,), 7.0376836292e-2, jnp.float32)
    for cc in (-1.1514610310e-1, 1.1676998740e-1, -1.2420140846e-1,
               1.4249322787e-1, -1.6668057665e-1, 2.0000714765e-1,
               -2.4999993993e-1, 3.3333331174e-1):
        p = p * z + cc
    y = z * zz * p
    y = y + ef * (-2.12194440e-4)
    y = y - 0.5 * zz
    return z + y + ef * 0.693359375


def _sc_body(pred_hbm, tgt_hbm, out_hbm, bufa, bufb, tgtv, partv, sema, semb):
    cid = lax.axis_index("c")
    sid = lax.axis_index("s")
    wid = sid * _NC + cid
    row0 = wid * _RPW
    pltpu.sync_copy(tgt_hbm.at[pl.ds(row0, _RPW)], tgtv)
    iota = lax.iota(jnp.int32, 16)

    def start_fetch(c, buf, sem):
        pltpu.async_copy(pred_hbm.at[pl.ds(row0 + c * _RCH, _RCH)], buf, sem)

    def wait_fetch(buf, sem):
        pltpu.make_async_copy(
            pred_hbm.at[pl.ds(0, _RCH)], buf, sem).wait()

    def process16(buf, c, g, lsum):
        rows = g * 16 + iota
        tv = tgtv[pl.ds(c * _RCH + g * 16, 16)]

        col0 = jnp.zeros((16,), jnp.int32)

        def p1(_, carry):
            m, col = carry
            for _u in range(_CU):
                v = plsc.load_gather(buf, [rows, col])
                m = jnp.maximum(m, v)
                col = col + 1
            return (m, col)

        m, _ = lax.fori_loop(
            0, _CB, p1, (jnp.full((16,), -3.0e38, jnp.float32), col0))

        def p2(_, carry):
            s, col = carry
            for _u in range(_CU):
                v = plsc.load_gather(buf, [rows, col])
                s = s + jnp.exp(v - m)
                col = col + 1
            return (s, col)

        s, _ = lax.fori_loop(
            0, _CB, p2, (jnp.zeros((16,), jnp.float32), col0))

        G = jnp.zeros((16,), jnp.float32)
        S = jnp.zeros((16,), jnp.float32)
        for d in range(-3, 4):
            w = 1.0 if d == 0 else _DEC[abs(d)]
            pos = tv + d
            valid = (pos >= 0) & (pos < _NCLS)
            posc = jnp.minimum(jnp.maximum(pos, 0), _NCLS - 1)
            gv = plsc.load_gather(buf, [rows, posc])
            G = G + jnp.where(valid, gv * w, 0.0)
            S = S + jnp.where(valid, jnp.full((16,), w, jnp.float32), 0.0)

        return lsum + (S * (m + _log16(s)) - G)

    start_fetch(0, bufa, sema)

    def pair_body(i, lsum):
        ca = 2 * i
        wait_fetch(bufa, sema)
        start_fetch(ca + 1, bufb, semb)
        for g in range(_RCH // 16):
            lsum = process16(bufa, ca, g, lsum)
        wait_fetch(bufb, semb)
        nxt = jnp.minimum(ca + 2, _NCHUNK - 1)
        start_fetch(nxt, bufa, sema)
        for g in range(_RCH // 16):
            lsum = process16(bufb, ca + 1, g, lsum)
        return lsum

    lsum = lax.fori_loop(
        0, _NCHUNK // 2, pair_body, jnp.zeros((16,), jnp.float32))
    wait_fetch(bufa, sema)
    partv[...] = lsum
    pltpu.sync_copy(partv, out_hbm.at[pl.ds(wid * 16, 16)])


_sc_call = functools.partial(
    pl.kernel,
    out_type=jax.ShapeDtypeStruct((_NW * 16,), jnp.float32),
    mesh=plsc.VectorSubcoreMesh(core_axis_name="c", subcore_axis_name="s"),
    compiler_params=pltpu.CompilerParams(
        needs_layout_passes=False, use_tc_tiling_on_sc=True),
    scratch_types=[
        pltpu.VMEM((_RCH, _NCLS), jnp.float32),
        pltpu.VMEM((_RCH, _NCLS), jnp.float32),
        pltpu.VMEM((_RPW,), jnp.int32),
        pltpu.VMEM((16,), jnp.float32),
        pltpu.SemaphoreType.DMA,
        pltpu.SemaphoreType.DMA,
    ],
)(_sc_body)


def _fin_body(p_ref, o_ref):
    o_ref[0, 0] = jnp.sum(p_ref[...]) * (1.0 / _ROWS)


_fin = pl.pallas_call(
    _fin_body,
    out_shape=jax.ShapeDtypeStruct((1, 1), jnp.float32),
    out_specs=pl.BlockSpec(memory_space=pltpu.SMEM),
)


def kernel(prediction, target):
    pred2 = prediction.reshape(_ROWS, _NCLS)
    tgtf = target.reshape(_ROWS)
    parts = _sc_call(pred2, tgtf)
    return _fin(parts.reshape(_NW, 16))[0, 0]


# speedup vs baseline: 2.7770x; 2.7770x over previous
import functools
import math

import jax
import jax.numpy as jnp
from jax import lax
from jax.experimental import pallas as pl
from jax.experimental.pallas import tpu as pltpu
from jax.experimental.pallas import tpu_sc as plsc

_NCLS = 722
_DEC = [math.exp(-(2.0 ** d) / 4.0) for d in range(4)]
_NC, _NS = 2, 16
_NW = _NC * _NS
_ROWS = 16 * 2048
_RPW = _ROWS // _NW
_RCH = 64
_NCHUNK = _RPW // _RCH
_CB, _CU = 38, 19


def _log16(x):
    bits = plsc.bitcast(x, jnp.int32)
    e = ((bits >> 23) & 0xFF) - 126
    f = plsc.bitcast((bits & 0x7FFFFF) | 0x3F000000, jnp.float32)
    ef = e.astype(jnp.float32)
    big = f > 0.70710678118654752440
    z = jnp.where(big, f - 1.0, 2.0 * f - 1.0)
    ef = jnp.where(big, ef, ef - 1.0)
    zz = z * z
    p = jnp.full((16,), 7.0376836292e-2, jnp.float32)
    for cc in (-1.1514610310e-1, 1.1676998740e-1, -1.2420140846e-1,
               1.4249322787e-1, -1.6668057665e-1, 2.0000714765e-1,
               -2.4999993993e-1, 3.3333331174e-1):
        p = p * z + cc
    y = z * zz * p
    y = y + ef * (-2.12194440e-4)
    y = y - 0.5 * zz
    return z + y + ef * 0.693359375


def _sc_body(pred_hbm, tgt_hbm, out_hbm, bufa, bufb, tgtv, partv, sema, semb):
    cid = lax.axis_index("c")
    sid = lax.axis_index("s")
    wid = sid * _NC + cid
    row0 = wid * _RPW
    pltpu.sync_copy(tgt_hbm.at[pl.ds(row0, _RPW)], tgtv)
    iota = lax.iota(jnp.int32, 16)

    def start_fetch(c, buf, sem):
        pltpu.async_copy(
            pred_hbm.at[pl.ds((row0 + c * _RCH) * _NCLS, _RCH * _NCLS)],
            buf, sem)

    def wait_fetch(buf, sem):
        pltpu.make_async_copy(
            pred_hbm.at[pl.ds(0, _RCH * _NCLS)], buf, sem).wait()

    _NG = _RCH // 16

    def p1_loop(buf, off0):
        def p1(_, carry):
            m, idx = carry
            for _u in range(_CU):
                v = plsc.load_gather(buf, [idx])
                m = jnp.maximum(m, v)
                idx = idx + 1
            return (m, idx)

        m, _ = lax.fori_loop(
            0, _CB, p1, (jnp.full((16,), -3.0e38, jnp.float32), off0))
        return m

    def p2_loop(buf, off0, m):
        def p2(_, carry):
            s, idx = carry
            for _u in range(_CU):
                v = plsc.load_gather(buf, [idx])
                s = s + jnp.exp(v - m)
                idx = idx + 1
            return (s, idx)

        s, _ = lax.fori_loop(
            0, _CB, p2, (jnp.zeros((16,), jnp.float32), off0))
        return s

    def fused_loop(buf, offa, ma, offb):
        def body(_, carry):
            sa, idxa, mb, idxb = carry
            for _u in range(_CU):
                va = plsc.load_gather(buf, [idxa])
                sa = sa + jnp.exp(va - ma)
                idxa = idxa + 1
                vb = plsc.load_gather(buf, [idxb])
                mb = jnp.maximum(mb, vb)
                idxb = idxb + 1
            return (sa, idxa, mb, idxb)

        sa, _, mb, _ = lax.fori_loop(
            0, _CB, body,
            (jnp.zeros((16,), jnp.float32), offa,
             jnp.full((16,), -3.0e38, jnp.float32), offb))
        return sa, mb

    def finish16(buf, c, g, off0, m, s, lsum):
        tv = tgtv[pl.ds(c * _RCH + g * 16, 16)]
        G = jnp.zeros((16,), jnp.float32)
        S = jnp.zeros((16,), jnp.float32)
        for d in range(-3, 4):
            w = 1.0 if d == 0 else _DEC[abs(d)]
            pos = tv + d
            valid = (pos >= 0) & (pos < _NCLS)
            posc = jnp.minimum(jnp.maximum(pos, 0), _NCLS - 1)
            gv = plsc.load_gather(buf, [off0 + posc])
            G = G + jnp.where(valid, gv * w, 0.0)
            S = S + jnp.where(valid, jnp.full((16,), w, jnp.float32), 0.0)
        return lsum + (S * (m + _log16(s)) - G)

    def process_chunk(buf, c, lsum):
        offs = [(g * 16 + iota) * _NCLS for g in range(_NG)]
        ms = [p1_loop(buf, offs[0])]
        ss = []
        for g in range(_NG - 1):
            sa, mb = fused_loop(buf, offs[g], ms[g], offs[g + 1])
            ss.append(sa)
            ms.append(mb)
        ss.append(p2_loop(buf, offs[_NG - 1], ms[_NG - 1]))
        for g in range(_NG):
            lsum = finish16(buf, c, g, offs[g], ms[g], ss[g], lsum)
        return lsum

    start_fetch(0, bufa, sema)

    def pair_body(i, lsum):
        ca = 2 * i
        wait_fetch(bufa, sema)
        start_fetch(ca + 1, bufb, semb)
        lsum = process_chunk(bufa, ca, lsum)
        wait_fetch(bufb, semb)
        nxt = jnp.minimum(ca + 2, _NCHUNK - 1)
        start_fetch(nxt, bufa, sema)
        lsum = process_chunk(bufb, ca + 1, lsum)
        return lsum

    lsum = lax.fori_loop(
        0, _NCHUNK // 2, pair_body, jnp.zeros((16,), jnp.float32))
    wait_fetch(bufa, sema)
    partv[...] = lsum
    pltpu.sync_copy(partv, out_hbm.at[pl.ds(wid * 16, 16)])


_sc_call = functools.partial(
    pl.kernel,
    out_type=jax.ShapeDtypeStruct((_NW * 16,), jnp.float32),
    mesh=plsc.VectorSubcoreMesh(core_axis_name="c", subcore_axis_name="s"),
    compiler_params=pltpu.CompilerParams(needs_layout_passes=False),
    scratch_types=[
        pltpu.VMEM((_RCH * _NCLS,), jnp.float32),
        pltpu.VMEM((_RCH * _NCLS,), jnp.float32),
        pltpu.VMEM((_RPW,), jnp.int32),
        pltpu.VMEM((16,), jnp.float32),
        pltpu.SemaphoreType.DMA,
        pltpu.SemaphoreType.DMA,
    ],
)(_sc_body)


def _fin_body(p_ref, o_ref):
    o_ref[0, 0] = jnp.sum(p_ref[...]) * (1.0 / _ROWS)


_fin = pl.pallas_call(
    _fin_body,
    out_shape=jax.ShapeDtypeStruct((1, 1), jnp.float32),
    out_specs=pl.BlockSpec(memory_space=pltpu.SMEM),
)


def kernel(prediction, target):
    predf = prediction.reshape(_ROWS * _NCLS)
    tgtf = target.reshape(_ROWS)
    parts = _sc_call(predf, tgtf)
    return _fin(parts.reshape(_NW, 16))[0, 0]
